# Initial kernel scaffold; baseline (speedup 1.0000x reference)
import jax
import jax.numpy as jnp
from jax.experimental import pallas as pl


def kernel(s_L, v_L, pos_L, s_P, v_P, pos_P, Wq, bq, Wk, bk, Wv, bv):
    raise NotImplementedError("write your pallas kernel here")


# speedup vs baseline: 2.1674x; 2.1674x over previous
import jax
import jax.numpy as jnp
from jax.experimental import pallas as pl
from jax.experimental.pallas import tpu as pltpu

N_L = 2048
N_P = 8192
DIM = 256
R2 = 100.0

BP = 1024
C = N_P // BP
NEG = -1e30


def _body(sL_ref, A_ref, B_ref, sP_ref,
          Wq_ref, bq_ref, Wk_ref, bk_ref, Wv_ref, bv_ref,
          out_ref, q_ref, acc_ref, mz_ref):
    j = pl.program_id(0)

    @pl.when(j == 0)
    def _init():
        q_ref[...] = (jnp.dot(sL_ref[...], Wq_ref[...].T,
                              preferred_element_type=jnp.float32)
                      + bq_ref[...])
        acc_ref[...] = jnp.zeros_like(acc_ref)
        mz_ref[0] = NEG
        mz_ref[1] = 0.0

    k = (jnp.dot(sP_ref[...], Wk_ref[...].T,
                 preferred_element_type=jnp.float32) + bk_ref[...])
    v = (jnp.dot(sP_ref[...], Wv_ref[...].T,
                 preferred_element_type=jnp.float32) + bv_ref[...])
    d2 = jnp.dot(A_ref[...], B_ref[...].T, preferred_element_type=jnp.float32)
    mask = d2 <= R2
    logits = jnp.dot(q_ref[...], k.T,
                     preferred_element_type=jnp.float32) * 0.125
    bm = jnp.max(jnp.where(mask, logits, NEG))
    m_old = mz_ref[0]
    m_new = jnp.maximum(m_old, bm)
    scale = jnp.exp(m_old - m_new)
    e = jnp.where(mask, jnp.exp(logits - m_new), 0.0)
    mz_ref[1] = mz_ref[1] * scale + jnp.sum(e)
    acc_ref[...] = (acc_ref[...] * scale
                    + jnp.dot(e, v, preferred_element_type=jnp.float32))
    mz_ref[0] = m_new

    @pl.when(j == C - 1)
    def _final():
        Z = mz_ref[1]
        Zs = jnp.where(Z > 0.0, Z, 1.0)
        out_ref[...] = sL_ref[...] + acc_ref[...] * (1.0 / Zs)


def _attend(s_L, A, B, s_P, Wq, bq, Wk, bk, Wv, bv, interpret=False):
    grid = (C,)
    res = lambda i: (0, 0)
    col = lambda i: (i, 0)
    out = pl.pallas_call(
        _body,
        grid=grid,
        in_specs=[
            pl.BlockSpec((N_L, DIM), res),
            pl.BlockSpec((N_L, 8), res),
            pl.BlockSpec((BP, 8), col),
            pl.BlockSpec((BP, DIM), col),
            pl.BlockSpec((DIM, DIM), res),
            pl.BlockSpec((1, DIM), res),
            pl.BlockSpec((DIM, DIM), res),
            pl.BlockSpec((1, DIM), res),
            pl.BlockSpec((DIM, DIM), res),
            pl.BlockSpec((1, DIM), res),
        ],
        out_specs=pl.BlockSpec((N_L, DIM), res),
        out_shape=jax.ShapeDtypeStruct((N_L, DIM), jnp.float32),
        scratch_shapes=[
            pltpu.VMEM((N_L, DIM), jnp.float32),
            pltpu.VMEM((N_L, DIM), jnp.float32),
            pltpu.SMEM((2,), jnp.float32),
        ],
        interpret=interpret,
    )(s_L, A, B, s_P, Wq, bq, Wk, bk, Wv, bv)
    return out


def kernel(s_L, v_L, pos_L, s_P, v_P, pos_P, Wq, bq, Wk, bk, Wv, bv):
    nL = jnp.sum(pos_L * pos_L, axis=1, keepdims=True)
    nP = jnp.sum(pos_P * pos_P, axis=1, keepdims=True)
    oneL = jnp.ones_like(nL)
    oneP = jnp.ones_like(nP)
    zL = jnp.zeros((N_L, 3), jnp.float32)
    zP = jnp.zeros((N_P, 3), jnp.float32)
    A = jnp.concatenate([pos_L, nL, oneL, zL], axis=1)
    B = jnp.concatenate([-2.0 * pos_P, oneP, nP, zP], axis=1)
    s_L_out = _attend(s_L, A, B, s_P, Wq, bq.reshape(1, DIM),
                      Wk, bk.reshape(1, DIM), Wv, bv.reshape(1, DIM))
    return (s_L_out, v_L)
